# Initial kernel scaffold; baseline (speedup 1.0000x reference)
import functools
import jax
import jax.numpy as jnp
from jax import lax
from jax.experimental import pallas as pl
from jax.experimental.pallas import tpu as pltpu
from jax.experimental.pallas import tpu_sc as plsc

N = 10000
D = 128
C = 128
NPAD = 10240
NC, NS, L = 2, 16, 16
NW = NC * NS
EB = 128
E_TOT = 320000 + N
NBLK = -(-E_TOT // (NW * EB))
EW = NBLK * EB
EPAD = EW * NW
RPS = NPAD // NS
BLK = 512


def _pre_body(z_ref, wl_ref, wr_ref, bl_ref, br_ref, xl_ref, xr_ref):
    zb = z_ref[...]
    xl_ref[...] = jnp.dot(zb, wl_ref[...], preferred_element_type=jnp.float32) + bl_ref[...]
    xr_ref[...] = jnp.dot(zb, wr_ref[...], preferred_element_type=jnp.float32) + br_ref[...]


_pre_call = pl.pallas_call(
    _pre_body,
    grid=(NPAD // BLK,),
    in_specs=[
        pl.BlockSpec((BLK, D), lambda i: (i, 0)),
        pl.BlockSpec((D, C), lambda i: (0, 0)),
        pl.BlockSpec((D, C), lambda i: (0, 0)),
        pl.BlockSpec((1, C), lambda i: (0, 0)),
        pl.BlockSpec((1, C), lambda i: (0, 0)),
    ],
    out_specs=[
        pl.BlockSpec((BLK, C), lambda i: (i, 0)),
        pl.BlockSpec((BLK, C), lambda i: (i, 0)),
    ],
    out_shape=[jax.ShapeDtypeStruct((NPAD, C), jnp.float32)] * 2,
)


def _post_body(o0_ref, o1_ref, d0_ref, d1_ref, bg_ref, wm_ref, bm_ref, wv_ref,
               bv_ref, eps_ref, mean_ref, var_ref, samp_ref):
    s = o0_ref[...] + o1_ref[...]
    den = d0_ref[...] + d1_ref[...]
    feats = s / (den[:, 0:1] + 1e-16) + bg_ref[...]
    mean = jnp.dot(feats, wm_ref[...], preferred_element_type=jnp.float32) + bm_ref[...]
    logv = jnp.dot(feats, wv_ref[...], preferred_element_type=jnp.float32) + bv_ref[...]
    var = jnp.exp(logv) + 1e-4
    mean_ref[...] = mean
    var_ref[...] = var
    samp_ref[...] = mean + jnp.sqrt(var) * eps_ref[...]


_post_call = pl.pallas_call(
    _post_body,
    grid=(NPAD // BLK,),
    in_specs=[
        pl.BlockSpec((BLK, C), lambda i: (i, 0)),
        pl.BlockSpec((BLK, C), lambda i: (i, 0)),
        pl.BlockSpec((BLK, L), lambda i: (i, 0)),
        pl.BlockSpec((BLK, L), lambda i: (i, 0)),
        pl.BlockSpec((1, C), lambda i: (0, 0)),
        pl.BlockSpec((C, C), lambda i: (0, 0)),
        pl.BlockSpec((1, C), lambda i: (0, 0)),
        pl.BlockSpec((C, C), lambda i: (0, 0)),
        pl.BlockSpec((1, C), lambda i: (0, 0)),
        pl.BlockSpec((BLK, C), lambda i: (i, 0)),
    ],
    out_specs=[pl.BlockSpec((BLK, C), lambda i: (i, 0))] * 3,
    out_shape=[jax.ShapeDtypeStruct((NPAD, C), jnp.float32)] * 3,
)


def _edge_body(xl_hbm, xr_hbm, src_hbm, dst_hbm, att_hbm, zc_hbm, zl_hbm,
               out_hbm, den_hbm,
               src_v, dst_v, xl_v, xr_v, den_v, att_v, out_sh, den_sh,
               sem1, sem2):
    cid = lax.axis_index("c")
    sid = lax.axis_index("s")
    w = cid * NS + sid

    pltpu.sync_copy(zc_hbm.at[pl.ds(sid * RPS, RPS)], out_sh.at[pl.ds(sid * RPS, RPS)])
    pltpu.sync_copy(zl_hbm.at[pl.ds(sid * RPS, RPS)], den_sh.at[pl.ds(sid * RPS, RPS)])
    pltpu.sync_copy(att_hbm, att_v)
    plsc.subcore_barrier()

    base = w * EW

    def block(b, carry):
        eb = base + b * EB
        pltpu.sync_copy(src_hbm.at[pl.ds(eb, EB)], src_v)
        pltpu.sync_copy(dst_hbm.at[pl.ds(eb, EB)], dst_v)
        cp1 = pltpu.async_copy(xl_hbm.at[src_v], xl_v, sem1)
        cp2 = pltpu.async_copy(xr_hbm.at[dst_v], xr_v, sem2)
        cp1.wait()
        cp2.wait()

        lane = lax.iota(jnp.int32, L)

        def edge(e, carry2):
            acc = jnp.zeros((L,), jnp.float32)
            for c in range(C // L):
                v = xl_v[e, pl.ds(c * L, L)] + xr_v[e, pl.ds(c * L, L)]
                v = jnp.maximum(v, 0.2 * v)
                acc = acc + att_v[pl.ds(c * L, L)] * v
            ex = jnp.exp(jnp.full((L,), jnp.sum(acc), jnp.float32))
            for c in range(C // L):
                xl_v[e, pl.ds(c * L, L)] = ex * xl_v[e, pl.ds(c * L, L)]
            den_v[e, :] = jnp.where(lane == 0, ex, 0.0)
            return carry2

        lax.fori_loop(0, EB, edge, 0)
        pltpu.sync_copy(xl_v, out_sh.at[dst_v], add=True)
        pltpu.sync_copy(den_v, den_sh.at[dst_v], add=True)
        return carry

    lax.fori_loop(0, NBLK, block, 0)
    plsc.subcore_barrier()

    pltpu.sync_copy(out_sh.at[pl.ds(sid * RPS, RPS)], out_hbm.at[cid, pl.ds(sid * RPS, RPS)])
    pltpu.sync_copy(den_sh.at[pl.ds(sid * RPS, RPS)], den_hbm.at[cid, pl.ds(sid * RPS, RPS)])


_edge_call = functools.partial(
    pl.kernel,
    out_type=(
        jax.ShapeDtypeStruct((NC, NPAD, C), jnp.float32),
        jax.ShapeDtypeStruct((NC, NPAD, L), jnp.float32),
    ),
    mesh=plsc.VectorSubcoreMesh(core_axis_name="c", subcore_axis_name="s",
                                num_cores=NC, num_subcores=NS),
    scratch_types=[
        pltpu.VMEM((EB,), jnp.int32),
        pltpu.VMEM((EB,), jnp.int32),
        pltpu.VMEM((EB, C), jnp.float32),
        pltpu.VMEM((EB, C), jnp.float32),
        pltpu.VMEM((EB, L), jnp.float32),
        pltpu.VMEM((C,), jnp.float32),
        pltpu.VMEM_SHARED((NPAD, C), jnp.float32),
        pltpu.VMEM_SHARED((NPAD, L), jnp.float32),
        pltpu.SemaphoreType.DMA,
        pltpu.SemaphoreType.DMA,
    ],
)(_edge_body)


def kernel(z, edge_index, W_l, b_l, W_r, b_r, att, bias_gat, W_m, b_m, W_v, b_v):
    zp = jnp.pad(z, ((0, NPAD - N), (0, 0)))
    xl, xr = _pre_call(zp, W_l, W_r, b_l.reshape(1, C), b_r.reshape(1, C))

    sl = jnp.arange(N, dtype=jnp.int32)
    fill = jnp.full((EPAD - E_TOT,), N, jnp.int32)
    src = jnp.concatenate([edge_index[0], sl, fill])
    dst = jnp.concatenate([edge_index[1], sl, fill])

    out_p, den_p = _edge_call(
        xl, xr, src, dst, att.reshape(C),
        jnp.zeros((NPAD, C), jnp.float32), jnp.zeros((NPAD, L), jnp.float32))

    eps = jax.random.normal(jax.random.key(42), (N, C), jnp.float32)
    epsp = jnp.pad(eps, ((0, NPAD - N), (0, 0)))
    mean, var, samp = _post_call(
        out_p[0], out_p[1], den_p[0], den_p[1], bias_gat.reshape(1, C),
        W_m, b_m.reshape(1, C), W_v, b_v.reshape(1, C), epsp)
    return mean[:N], var[:N], samp[:N]


# speedup vs baseline: 6.2114x; 6.2114x over previous
import functools
import jax
import jax.numpy as jnp
from jax import lax
from jax.experimental import pallas as pl
from jax.experimental.pallas import tpu as pltpu
from jax.experimental.pallas import tpu_sc as plsc


def _permute(v, idx):
    return lax.gather(
        v, idx[:, None],
        lax.GatherDimensionNumbers(offset_dims=(), collapsed_slice_dims=(0,),
                                   start_index_map=(0,)),
        slice_sizes=(1,), mode=lax.GatherScatterMode.PROMISE_IN_BOUNDS)

N = 10000
D = 128
C = 128
NPAD = 10112
NC, NS, L = 2, 16, 16
NW = NC * NS
EB = 64
E_TOT = 320000 + N
NBLK = -(-E_TOT // (NW * EB))
EW = NBLK * EB
EPAD = EW * NW
NCH = NPAD // EB
NCHW = -(-NCH // NS)
BLK = 632


def _pre_body(z_ref, wl_ref, wr_ref, bl_ref, br_ref, xl_ref, xr_ref):
    zb = z_ref[...]
    xl_ref[...] = jnp.dot(zb, wl_ref[...], preferred_element_type=jnp.float32) + bl_ref[...]
    xr_ref[...] = jnp.dot(zb, wr_ref[...], preferred_element_type=jnp.float32) + br_ref[...]


_pre_call = pl.pallas_call(
    _pre_body,
    grid=(NPAD // BLK,),
    in_specs=[
        pl.BlockSpec((BLK, D), lambda i: (i, 0)),
        pl.BlockSpec((D, C), lambda i: (0, 0)),
        pl.BlockSpec((D, C), lambda i: (0, 0)),
        pl.BlockSpec((1, C), lambda i: (0, 0)),
        pl.BlockSpec((1, C), lambda i: (0, 0)),
    ],
    out_specs=[
        pl.BlockSpec((BLK, C), lambda i: (i, 0)),
        pl.BlockSpec((BLK, C), lambda i: (i, 0)),
    ],
    out_shape=[jax.ShapeDtypeStruct((NPAD, C), jnp.float32)] * 2,
)


def _post_body(n0_ref, n1_ref, d0_ref, d1_ref, bg_ref, wm_ref, bm_ref, wv_ref,
               bv_ref, eps_ref, mean_ref, var_ref, samp_ref):
    s = n0_ref[...] + n1_ref[...]
    den = d0_ref[...] + d1_ref[...]
    feats = s / (den[:, 0:1] + 1e-16) + bg_ref[...]
    mean = jnp.dot(feats, wm_ref[...], preferred_element_type=jnp.float32) + bm_ref[...]
    logv = jnp.dot(feats, wv_ref[...], preferred_element_type=jnp.float32) + bv_ref[...]
    var = jnp.exp(logv) + 1e-4
    mean_ref[...] = mean
    var_ref[...] = var
    samp_ref[...] = mean + jnp.sqrt(var) * eps_ref[...]


_post_call = pl.pallas_call(
    _post_body,
    grid=(NPAD // BLK,),
    in_specs=[
        pl.BlockSpec((BLK, C), lambda i: (i, 0)),
        pl.BlockSpec((BLK, C), lambda i: (i, 0)),
        pl.BlockSpec((BLK, C), lambda i: (i, 0)),
        pl.BlockSpec((BLK, C), lambda i: (i, 0)),
        pl.BlockSpec((1, C), lambda i: (0, 0)),
        pl.BlockSpec((C, C), lambda i: (0, 0)),
        pl.BlockSpec((1, C), lambda i: (0, 0)),
        pl.BlockSpec((C, C), lambda i: (0, 0)),
        pl.BlockSpec((1, C), lambda i: (0, 0)),
        pl.BlockSpec((BLK, C), lambda i: (i, 0)),
    ],
    out_specs=[pl.BlockSpec((BLK, C), lambda i: (i, 0))] * 3,
    out_shape=[jax.ShapeDtypeStruct((NPAD, C), jnp.float32)] * 3,
)


def _edge_body(xl_hbm, xr_hbm, src_hbm, dst_hbm, att_hbm,
               num_hbm, den_hbm, exe_hbm,
               src_v, dst_v, dst2_v, idx2_v, xl_v, xr_v, msg_v, ex_v, att_v,
               acc_sh, sem1, sem2):
    cid = lax.axis_index("c")
    sid = lax.axis_index("s")
    w = cid * NS + sid
    lane = lax.iota(jnp.int32, L)
    zv = jnp.zeros((L,), jnp.float32)
    base = w * EW

    def zrow(r, carry):
        for c in range(C // L):
            msg_v[r, pl.ds(c * L, L)] = zv
        return carry

    def zinit(j, carry):
        ch = sid + j * NS

        @pl.when(ch < NCH)
        def _():
            for c in range(EB // L):
                idx2_v[0, pl.ds(c * L, L)] = ch * EB + c * L + lane
            pltpu.sync_copy(msg_v, acc_sh.at[idx2_v.at[0]])

        return carry

    def readback(j, out_ref):
        ch = sid + j * NS

        @pl.when(ch < NCH)
        def _():
            for c in range(EB // L):
                idx2_v[0, pl.ds(c * L, L)] = ch * EB + c * L + lane
            pltpu.async_copy(acc_sh.at[idx2_v.at[0]], msg_v, sem1).wait()
            pltpu.sync_copy(msg_v, out_ref.at[pl.ds(cid * NPAD + ch * EB, EB)])

    lax.fori_loop(0, EB, zrow, 0)
    lax.fori_loop(0, NCHW, zinit, 0)
    pltpu.sync_copy(att_hbm, att_v)
    plsc.subcore_barrier()

    def block(b, carry):
        eb = base + b * EB
        pltpu.sync_copy(src_hbm.at[pl.ds(eb, EB)], src_v)
        pltpu.sync_copy(dst_hbm.at[pl.ds(eb, EB)], dst_v)
        pltpu.sync_copy(dst_hbm.at[pl.ds(eb, EB)], dst2_v.at[0])
        cp1 = pltpu.async_copy(xl_hbm.at[src_v], xl_v, sem1)
        cp2 = pltpu.async_copy(xr_hbm.at[dst_v], xr_v, sem2)
        cp1.wait()
        cp2.wait()

        def edge(e, carry2):
            acc = jnp.zeros((L,), jnp.float32)
            for c in range(C // L):
                v = xl_v[e, pl.ds(c * L, L)] + xr_v[e, pl.ds(c * L, L)]
                v = jnp.maximum(v, 0.2 * v)
                acc = acc + att_v[pl.ds(c * L, L)] * v
            for k2 in (8, 4, 2, 1):
                acc = acc + _permute(acc, jnp.bitwise_xor(lane, k2))
            ex = jnp.exp(acc)
            for c in range(C // L):
                msg_v[e, pl.ds(c * L, L)] = ex * xl_v[e, pl.ds(c * L, L)]
            ex_v[e, :] = ex
            return carry2

        lax.fori_loop(0, EB, edge, 0)
        pltpu.sync_copy(msg_v, acc_sh.at[dst2_v.at[0]], add=True)
        pltpu.sync_copy(ex_v, exe_hbm.at[pl.ds(eb, EB)])
        return carry

    lax.fori_loop(0, NBLK, block, 0)
    plsc.subcore_barrier()

    lax.fori_loop(0, NCHW, lambda j, c: (readback(j, num_hbm), c)[1], 0)
    plsc.subcore_barrier()

    lax.fori_loop(0, EB, zrow, 0)
    lax.fori_loop(0, NCHW, zinit, 0)
    plsc.subcore_barrier()

    def dblock(b, carry):
        eb = base + b * EB
        pltpu.sync_copy(dst_hbm.at[pl.ds(eb, EB)], dst2_v.at[0])
        pltpu.sync_copy(exe_hbm.at[pl.ds(eb, EB)], ex_v)

        def drow(e, carry2):
            ex = ex_v[e, :]
            for c in range(C // L):
                msg_v[e, pl.ds(c * L, L)] = ex
            return carry2

        lax.fori_loop(0, EB, drow, 0)
        pltpu.sync_copy(msg_v, acc_sh.at[dst2_v.at[0]], add=True)
        return carry

    lax.fori_loop(0, NBLK, dblock, 0)
    plsc.subcore_barrier()

    lax.fori_loop(0, NCHW, lambda j, c: (readback(j, den_hbm), c)[1], 0)


_edge_call = functools.partial(
    pl.kernel,
    out_type=(
        jax.ShapeDtypeStruct((NC * NPAD, C), jnp.float32),
        jax.ShapeDtypeStruct((NC * NPAD, C), jnp.float32),
        jax.ShapeDtypeStruct((EPAD, L), jnp.float32),
    ),
    mesh=plsc.VectorSubcoreMesh(core_axis_name="c", subcore_axis_name="s",
                                num_cores=NC, num_subcores=NS),
    scratch_types=[
        pltpu.VMEM((EB,), jnp.int32),
        pltpu.VMEM((EB,), jnp.int32),
        pltpu.VMEM((1, EB), jnp.int32),
        pltpu.VMEM((1, EB), jnp.int32),
        pltpu.VMEM((EB, C), jnp.float32),
        pltpu.VMEM((EB, C), jnp.float32),
        pltpu.VMEM((EB, C), jnp.float32),
        pltpu.VMEM((EB, L), jnp.float32),
        pltpu.VMEM((C,), jnp.float32),
        pltpu.VMEM_SHARED((NPAD, C), jnp.float32),
        pltpu.SemaphoreType.DMA,
        pltpu.SemaphoreType.DMA,
    ],
)(_edge_body)


def kernel(z, edge_index, W_l, b_l, W_r, b_r, att, bias_gat, W_m, b_m, W_v, b_v):
    zp = jnp.pad(z, ((0, NPAD - N), (0, 0)))
    xl, xr = _pre_call(zp, W_l, W_r, b_l.reshape(1, C), b_r.reshape(1, C))

    sl = jnp.arange(N, dtype=jnp.int32)
    fill = jnp.full((EPAD - E_TOT,), N, jnp.int32)
    src = jnp.concatenate([edge_index[0], sl, fill])
    dst = jnp.concatenate([edge_index[1], sl, fill])

    num, den, _ = _edge_call(xl, xr, src, dst, att.reshape(C))

    eps = jax.random.normal(jax.random.key(42), (N, C), jnp.float32)
    epsp = jnp.pad(eps, ((0, NPAD - N), (0, 0)))
    mean, var, samp = _post_call(
        num[:NPAD], num[NPAD:], den[:NPAD], den[NPAD:],
        bias_gat.reshape(1, C),
        W_m, b_m.reshape(1, C), W_v, b_v.reshape(1, C), epsp)
    return mean[:N], var[:N], samp[:N]


# speedup vs baseline: 7.3849x; 1.1889x over previous
import functools
import jax
import jax.numpy as jnp
from jax import lax
from jax.experimental import pallas as pl
from jax.experimental.pallas import tpu as pltpu
from jax.experimental.pallas import tpu_sc as plsc


def _permute(v, idx):
    return lax.gather(
        v, idx[:, None],
        lax.GatherDimensionNumbers(offset_dims=(), collapsed_slice_dims=(0,),
                                   start_index_map=(0,)),
        slice_sizes=(1,), mode=lax.GatherScatterMode.PROMISE_IN_BOUNDS)

N = 10000
D = 128
C = 128
NPAD = 10112
DEN = NPAD // 8
NC, NS, L = 2, 16, 16
NW = NC * NS
EB = 64
E_TOT = 320000 + N
NBLK = -(-E_TOT // (NW * EB))
EW = NBLK * EB
EPAD = EW * NW
ACCR = -(-(NPAD + DEN) // EB) * EB
NCH = ACCR // EB
NCHW = -(-NCH // NS)
BLK = 632


def _pre_body(z_ref, wl_ref, wr_ref, bl_ref, br_ref, xl_ref, xr_ref):
    zb = z_ref[...]
    xl_ref[...] = jnp.dot(zb, wl_ref[...], preferred_element_type=jnp.float32) + bl_ref[...]
    xr_ref[...] = jnp.dot(zb, wr_ref[...], preferred_element_type=jnp.float32) + br_ref[...]


_pre_call = pl.pallas_call(
    _pre_body,
    grid=(NPAD // BLK,),
    in_specs=[
        pl.BlockSpec((BLK, D), lambda i: (i, 0)),
        pl.BlockSpec((D, C), lambda i: (0, 0)),
        pl.BlockSpec((D, C), lambda i: (0, 0)),
        pl.BlockSpec((1, C), lambda i: (0, 0)),
        pl.BlockSpec((1, C), lambda i: (0, 0)),
    ],
    out_specs=[
        pl.BlockSpec((BLK, C), lambda i: (i, 0)),
        pl.BlockSpec((BLK, C), lambda i: (i, 0)),
    ],
    out_shape=[jax.ShapeDtypeStruct((NPAD, C), jnp.float32)] * 2,
)


def _post_body(n0_ref, n1_ref, d0_ref, d1_ref, bg_ref, wm_ref, bm_ref, wv_ref,
               bv_ref, eps_ref, mean_ref, var_ref, samp_ref):
    s = n0_ref[...] + n1_ref[...]
    den = d0_ref[...] + d1_ref[...]
    feats = s / (den[:, 0:1] + 1e-16) + bg_ref[...]
    mean = jnp.dot(feats, wm_ref[...], preferred_element_type=jnp.float32) + bm_ref[...]
    logv = jnp.dot(feats, wv_ref[...], preferred_element_type=jnp.float32) + bv_ref[...]
    var = jnp.exp(logv) + 1e-4
    mean_ref[...] = mean
    var_ref[...] = var
    samp_ref[...] = mean + jnp.sqrt(var) * eps_ref[...]


_post_call = pl.pallas_call(
    _post_body,
    grid=(NPAD // BLK,),
    in_specs=[
        pl.BlockSpec((BLK, C), lambda i: (i, 0)),
        pl.BlockSpec((BLK, C), lambda i: (i, 0)),
        pl.BlockSpec((BLK, L), lambda i: (i, 0)),
        pl.BlockSpec((BLK, L), lambda i: (i, 0)),
        pl.BlockSpec((1, C), lambda i: (0, 0)),
        pl.BlockSpec((C, C), lambda i: (0, 0)),
        pl.BlockSpec((1, C), lambda i: (0, 0)),
        pl.BlockSpec((C, C), lambda i: (0, 0)),
        pl.BlockSpec((1, C), lambda i: (0, 0)),
        pl.BlockSpec((BLK, C), lambda i: (i, 0)),
    ],
    out_specs=[pl.BlockSpec((BLK, C), lambda i: (i, 0))] * 3,
    out_shape=[jax.ShapeDtypeStruct((NPAD, C), jnp.float32)] * 3,
)


def _edge_body(xl_hbm, xr_hbm, src_hbm, dst_hbm, att_hbm,
               acc_hbm,
               src_v, dst_v, dst2_v, dstp_v, idx2_v, xl_v, xr_v, msg_v, msg2_v,
               att_v, acc_sh, sem1, sem2):
    cid = lax.axis_index("c")
    sid = lax.axis_index("s")
    w = cid * NS + sid
    lane = lax.iota(jnp.int32, L)
    zv = jnp.zeros((L,), jnp.float32)
    base = w * EW

    def zrow(r, carry):
        for c in range(C // L):
            msg_v[r, pl.ds(c * L, L)] = zv
        return carry

    def zinit(j, carry):
        ch = sid + j * NS

        @pl.when(ch < NCH)
        def _():
            for c in range(EB // L):
                idx2_v[0, pl.ds(c * L, L)] = ch * EB + c * L + lane
            pltpu.sync_copy(msg_v, acc_sh.at[idx2_v.at[0]])

        return carry

    def readback(j, out_ref):
        ch = sid + j * NS

        @pl.when(ch < NCH)
        def _():
            for c in range(EB // L):
                idx2_v[0, pl.ds(c * L, L)] = ch * EB + c * L + lane
            pltpu.async_copy(acc_sh.at[idx2_v.at[0]], msg_v, sem1).wait()
            pltpu.sync_copy(msg_v, out_ref.at[pl.ds(cid * ACCR + ch * EB, EB)])

    lax.fori_loop(0, EB, zrow, 0)
    lax.fori_loop(0, NCHW, zinit, 0)
    pltpu.sync_copy(att_hbm, att_v)
    plsc.subcore_barrier()

    def block(b, carry):
        eb = base + b * EB
        pltpu.sync_copy(src_hbm.at[pl.ds(eb, EB)], src_v)
        pltpu.sync_copy(dst_hbm.at[pl.ds(eb, EB)], dst_v)
        pltpu.sync_copy(dst_hbm.at[pl.ds(eb, EB)], dst2_v.at[0])
        pltpu.sync_copy(dst_hbm.at[pl.ds(eb, EB)], dstp_v.at[0])
        cp1 = pltpu.async_copy(xl_hbm.at[src_v], xl_v, sem1)
        cp2 = pltpu.async_copy(xr_hbm.at[dst_v], xr_v, sem2)
        for c in range(EB // L):
            v = dstp_v[0, pl.ds(c * L, L)]
            dstp_v[0, pl.ds(c * L, L)] = lax.shift_right_logical(v, 3) + NPAD
        cp1.wait()
        cp2.wait()

        def edge(e, carry2):
            acc = jnp.zeros((L,), jnp.float32)
            for c in range(C // L):
                v = xl_v[e, pl.ds(c * L, L)] + xr_v[e, pl.ds(c * L, L)]
                v = jnp.maximum(v, 0.2 * v)
                acc = acc + att_v[pl.ds(c * L, L)] * v
            for k2 in (8, 4, 2, 1):
                acc = acc + _permute(acc, jnp.bitwise_xor(lane, k2))
            ex = jnp.exp(acc)
            g = jnp.bitwise_and(dst_v[pl.ds(e, 1)][0], 7)
            for c in range(C // L):
                msg_v[e, pl.ds(c * L, L)] = ex * xl_v[e, pl.ds(c * L, L)]
                msg2_v[e, pl.ds(c * L, L)] = jnp.where(g == c, ex, zv)
            return carry2

        lax.fori_loop(0, EB, edge, 0)
        pltpu.sync_copy(msg_v, acc_sh.at[dst2_v.at[0]], add=True)
        pltpu.sync_copy(msg2_v, acc_sh.at[dstp_v.at[0]], add=True)
        return carry

    lax.fori_loop(0, NBLK, block, 0)
    plsc.subcore_barrier()

    lax.fori_loop(0, NCHW, lambda j, c: (readback(j, acc_hbm), c)[1], 0)


_edge_call = functools.partial(
    pl.kernel,
    out_type=(
        jax.ShapeDtypeStruct((NC * ACCR, C), jnp.float32),
    ),
    mesh=plsc.VectorSubcoreMesh(core_axis_name="c", subcore_axis_name="s",
                                num_cores=NC, num_subcores=NS),
    scratch_types=[
        pltpu.VMEM((EB,), jnp.int32),
        pltpu.VMEM((EB,), jnp.int32),
        pltpu.VMEM((1, EB), jnp.int32),
        pltpu.VMEM((1, EB), jnp.int32),
        pltpu.VMEM((1, EB), jnp.int32),
        pltpu.VMEM((EB, C), jnp.float32),
        pltpu.VMEM((EB, C), jnp.float32),
        pltpu.VMEM((EB, C), jnp.float32),
        pltpu.VMEM((EB, C), jnp.float32),
        pltpu.VMEM((C,), jnp.float32),
        pltpu.VMEM_SHARED((ACCR, C), jnp.float32),
        pltpu.SemaphoreType.DMA,
        pltpu.SemaphoreType.DMA,
    ],
)(_edge_body)


def kernel(z, edge_index, W_l, b_l, W_r, b_r, att, bias_gat, W_m, b_m, W_v, b_v):
    zp = jnp.pad(z, ((0, NPAD - N), (0, 0)))
    xl, xr = _pre_call(zp, W_l, W_r, b_l.reshape(1, C), b_r.reshape(1, C))

    sl = jnp.arange(N, dtype=jnp.int32)
    fill = jnp.full((EPAD - E_TOT,), N, jnp.int32)
    src = jnp.concatenate([edge_index[0], sl, fill])
    dst = jnp.concatenate([edge_index[1], sl, fill])

    (acc,) = _edge_call(xl, xr, src, dst, att.reshape(C))

    den0 = acc[NPAD:NPAD + DEN].reshape(NPAD, L)
    den1 = acc[ACCR + NPAD:ACCR + NPAD + DEN].reshape(NPAD, L)

    eps = jax.random.normal(jax.random.key(42), (N, C), jnp.float32)
    epsp = jnp.pad(eps, ((0, NPAD - N), (0, 0)))
    mean, var, samp = _post_call(
        acc[:NPAD], acc[ACCR:ACCR + NPAD], den0, den1,
        bias_gat.reshape(1, C),
        W_m, b_m.reshape(1, C), W_v, b_v.reshape(1, C), epsp)
    return mean[:N], var[:N], samp[:N]


# speedup vs baseline: 9.2071x; 1.2468x over previous
import functools
import jax
import jax.numpy as jnp
from jax import lax
from jax.experimental import pallas as pl
from jax.experimental.pallas import tpu as pltpu
from jax.experimental.pallas import tpu_sc as plsc


def _permute(v, idx):
    return lax.gather(
        v, idx[:, None],
        lax.GatherDimensionNumbers(offset_dims=(), collapsed_slice_dims=(0,),
                                   start_index_map=(0,)),
        slice_sizes=(1,), mode=lax.GatherScatterMode.PROMISE_IN_BOUNDS)

N = 10000
D = 128
C = 128
NPAD = 10112
DEN = NPAD // 8
NC, NS, L = 2, 16, 16
NW = NC * NS
EB = 64
E_TOT = 320000 + N
NBLK = -(-E_TOT // (NW * EB))
EW = NBLK * EB
EPAD = EW * NW
ACCR = -(-(NPAD + DEN) // EB) * EB
NCH = ACCR // EB
NCHW = -(-NCH // NS)
BLK = 632


def _pre_body(z_ref, wl_ref, wr_ref, bl_ref, br_ref, xl_ref, xr_ref):
    zb = z_ref[...]
    xl_ref[...] = jnp.dot(zb, wl_ref[...], preferred_element_type=jnp.float32) + bl_ref[...]
    xr_ref[...] = jnp.dot(zb, wr_ref[...], preferred_element_type=jnp.float32) + br_ref[...]


_pre_call = pl.pallas_call(
    _pre_body,
    grid=(NPAD // BLK,),
    in_specs=[
        pl.BlockSpec((BLK, D), lambda i: (i, 0)),
        pl.BlockSpec((D, C), lambda i: (0, 0)),
        pl.BlockSpec((D, C), lambda i: (0, 0)),
        pl.BlockSpec((1, C), lambda i: (0, 0)),
        pl.BlockSpec((1, C), lambda i: (0, 0)),
    ],
    out_specs=[
        pl.BlockSpec((BLK, C), lambda i: (i, 0)),
        pl.BlockSpec((BLK, C), lambda i: (i, 0)),
    ],
    out_shape=[jax.ShapeDtypeStruct((NPAD, C), jnp.float32)] * 2,
)


def _post_body(n0_ref, n1_ref, d0_ref, d1_ref, bg_ref, wm_ref, bm_ref, wv_ref,
               bv_ref, eps_ref, mean_ref, var_ref, samp_ref):
    s = n0_ref[...] + n1_ref[...]
    den = d0_ref[...] + d1_ref[...]
    feats = s / (den[:, 0:1] + 1e-16) + bg_ref[...]
    mean = jnp.dot(feats, wm_ref[...], preferred_element_type=jnp.float32) + bm_ref[...]
    logv = jnp.dot(feats, wv_ref[...], preferred_element_type=jnp.float32) + bv_ref[...]
    var = jnp.exp(logv) + 1e-4
    mean_ref[...] = mean
    var_ref[...] = var
    samp_ref[...] = mean + jnp.sqrt(var) * eps_ref[...]


_post_call = pl.pallas_call(
    _post_body,
    grid=(NPAD // BLK,),
    in_specs=[
        pl.BlockSpec((BLK, C), lambda i: (i, 0)),
        pl.BlockSpec((BLK, C), lambda i: (i, 0)),
        pl.BlockSpec((BLK, L), lambda i: (i, 0)),
        pl.BlockSpec((BLK, L), lambda i: (i, 0)),
        pl.BlockSpec((1, C), lambda i: (0, 0)),
        pl.BlockSpec((C, C), lambda i: (0, 0)),
        pl.BlockSpec((1, C), lambda i: (0, 0)),
        pl.BlockSpec((C, C), lambda i: (0, 0)),
        pl.BlockSpec((1, C), lambda i: (0, 0)),
        pl.BlockSpec((BLK, C), lambda i: (i, 0)),
    ],
    out_specs=[pl.BlockSpec((BLK, C), lambda i: (i, 0))] * 3,
    out_shape=[jax.ShapeDtypeStruct((NPAD, C), jnp.float32)] * 3,
)


def _edge_body(xl_hbm, xr_hbm, src_hbm, dst_hbm, att_hbm,
               acc_hbm,
               src_a, src_b, dst_a, dst_b, dst2_a, dst2_b, dstp_a, dstp_b,
               idx2_v, xl_a, xl_b, xr_a, xr_b,
               att_v, acc_sh, sem_a, sem_b, sem1):
    msg_v = xl_a
    src_s, dst_s, dst2_s, dstp_s = (src_a, src_b), (dst_a, dst_b), \
        (dst2_a, dst2_b), (dstp_a, dstp_b)
    xl_s, xr_s, sem_s = (xl_a, xl_b), (xr_a, xr_b), (sem_a, sem_b)
    cid = lax.axis_index("c")
    sid = lax.axis_index("s")
    w = cid * NS + sid
    lane = lax.iota(jnp.int32, L)
    zv = jnp.zeros((L,), jnp.float32)
    base = w * EW

    def zrow(r, carry):
        for c in range(C // L):
            msg_v[r, pl.ds(c * L, L)] = zv
        return carry

    def zinit(j, carry):
        ch = sid + j * NS

        @pl.when(ch < NCH)
        def _():
            for c in range(EB // L):
                idx2_v[0, pl.ds(c * L, L)] = ch * EB + c * L + lane
            pltpu.sync_copy(msg_v, acc_sh.at[idx2_v.at[0]])

        return carry

    def readback(j, out_ref):
        ch = sid + j * NS

        @pl.when(ch < NCH)
        def _():
            for c in range(EB // L):
                idx2_v[0, pl.ds(c * L, L)] = ch * EB + c * L + lane
            pltpu.async_copy(acc_sh.at[idx2_v.at[0]], msg_v, sem1).wait()
            pltpu.sync_copy(msg_v, out_ref.at[pl.ds(cid * ACCR + ch * EB, EB)])

    lax.fori_loop(0, EB, zrow, 0)
    lax.fori_loop(0, NCHW, zinit, 0)
    pltpu.sync_copy(att_hbm, att_v)
    plsc.subcore_barrier()

    def gissue(i, s):
        eb = base + i * EB
        pltpu.sync_copy(src_hbm.at[pl.ds(eb, EB)], src_s[s])
        pltpu.sync_copy(dst_hbm.at[pl.ds(eb, EB)], dst_s[s])
        pltpu.sync_copy(dst_hbm.at[pl.ds(eb, EB)], dst2_s[s].at[0])
        pltpu.sync_copy(dst_hbm.at[pl.ds(eb, EB)], dstp_s[s].at[0])
        for c in range(EB // L):
            v = dstp_s[s][0, pl.ds(c * L, L)]
            dstp_s[s][0, pl.ds(c * L, L)] = lax.shift_right_logical(v, 3) + NPAD
        pltpu.async_copy(xl_hbm.at[src_s[s]], xl_s[s], sem_s[s])
        pltpu.async_copy(xr_hbm.at[dst_s[s]], xr_s[s], sem_s[s])

    gissue(0, 0)

    def pair(gi, carry):
        for s in range(2):
            i = gi * 2 + s
            ns = 1 - s

            @pl.when(i + 1 < NBLK)
            def _():
                gissue(i + 1, ns)

            pltpu.make_async_copy(xl_hbm.at[pl.ds(0, EB)], xl_s[s], sem_s[s]).wait()
            pltpu.make_async_copy(xr_hbm.at[pl.ds(0, EB)], xr_s[s], sem_s[s]).wait()
            xl_v, xr_v, dst_v = xl_s[s], xr_s[s], dst_s[s]

            def edge(e, carry2):
                acc = jnp.zeros((L,), jnp.float32)
                for c in range(C // L):
                    v = xl_v[e, pl.ds(c * L, L)] + xr_v[e, pl.ds(c * L, L)]
                    v = jnp.maximum(v, 0.2 * v)
                    acc = acc + att_v[pl.ds(c * L, L)] * v
                for k2 in (8, 4, 2, 1):
                    acc = acc + _permute(acc, jnp.bitwise_xor(lane, k2))
                ex = jnp.exp(acc)
                g = jnp.bitwise_and(dst_v[pl.ds(e, 1)][0], 7)
                for c in range(C // L):
                    xl_v[e, pl.ds(c * L, L)] = ex * xl_v[e, pl.ds(c * L, L)]
                    xr_v[e, pl.ds(c * L, L)] = jnp.where(g == c, ex, zv)
                return carry2

            lax.fori_loop(0, EB, edge, 0)
            pltpu.sync_copy(xl_v, acc_sh.at[dst2_s[s].at[0]], add=True)
            pltpu.sync_copy(xr_v, acc_sh.at[dstp_s[s].at[0]], add=True)
        return carry

    lax.fori_loop(0, NBLK // 2, pair, 0)
    plsc.subcore_barrier()

    lax.fori_loop(0, NCHW, lambda j, c: (readback(j, acc_hbm), c)[1], 0)


_edge_call = functools.partial(
    pl.kernel,
    out_type=(
        jax.ShapeDtypeStruct((NC * ACCR, C), jnp.float32),
    ),
    mesh=plsc.VectorSubcoreMesh(core_axis_name="c", subcore_axis_name="s",
                                num_cores=NC, num_subcores=NS),
    scratch_types=[
        pltpu.VMEM((EB,), jnp.int32),
        pltpu.VMEM((EB,), jnp.int32),
        pltpu.VMEM((EB,), jnp.int32),
        pltpu.VMEM((EB,), jnp.int32),
        pltpu.VMEM((1, EB), jnp.int32),
        pltpu.VMEM((1, EB), jnp.int32),
        pltpu.VMEM((1, EB), jnp.int32),
        pltpu.VMEM((1, EB), jnp.int32),
        pltpu.VMEM((1, EB), jnp.int32),
        pltpu.VMEM((EB, C), jnp.float32),
        pltpu.VMEM((EB, C), jnp.float32),
        pltpu.VMEM((EB, C), jnp.float32),
        pltpu.VMEM((EB, C), jnp.float32),
        pltpu.VMEM((C,), jnp.float32),
        pltpu.VMEM_SHARED((ACCR, C), jnp.float32),
        pltpu.SemaphoreType.DMA,
        pltpu.SemaphoreType.DMA,
        pltpu.SemaphoreType.DMA,
    ],
)(_edge_body)


def kernel(z, edge_index, W_l, b_l, W_r, b_r, att, bias_gat, W_m, b_m, W_v, b_v):
    zp = jnp.pad(z, ((0, NPAD - N), (0, 0)))
    xl, xr = _pre_call(zp, W_l, W_r, b_l.reshape(1, C), b_r.reshape(1, C))

    sl = jnp.arange(N, dtype=jnp.int32)
    fill = jnp.full((EPAD - E_TOT,), N, jnp.int32)
    src = jnp.concatenate([edge_index[0], sl, fill])
    dst = jnp.concatenate([edge_index[1], sl, fill])

    (acc,) = _edge_call(xl, xr, src, dst, att.reshape(C))

    den0 = acc[NPAD:NPAD + DEN].reshape(NPAD, L)
    den1 = acc[ACCR + NPAD:ACCR + NPAD + DEN].reshape(NPAD, L)

    eps = jax.random.normal(jax.random.key(42), (N, C), jnp.float32)
    epsp = jnp.pad(eps, ((0, NPAD - N), (0, 0)))
    mean, var, samp = _post_call(
        acc[:NPAD], acc[ACCR:ACCR + NPAD], den0, den1,
        bias_gat.reshape(1, C),
        W_m, b_m.reshape(1, C), W_v, b_v.reshape(1, C), epsp)
    return mean[:N], var[:N], samp[:N]
